# baseline (device time: 27338 ns/iter reference)
import jax
import jax.numpy as jnp
from jax import lax
from jax.experimental import pallas as pl
from jax.experimental.pallas import tpu as pltpu

N_DEV = 16
STEP_MASKS = [1, 3, 4, 8]
N_STEPS = len(STEP_MASKS)
B, Sq, Skv = 2, 128, 128
H_PER = 4
Dh = 64
D_MODEL = 512
QB = 64
CHUNKS = [(b, c) for b in range(B) for c in range(2)]


def kernel(x, Wq, K_ext, V_ext, Wo):
    i = lax.axis_index("i")
    K2 = K_ext.reshape(B, Skv, 64 * Dh)
    V2 = V_ext.reshape(B, Skv, 64 * Dh)
    K_loc = lax.dynamic_slice_in_dim(K2, i * H_PER * Dh, H_PER * Dh, axis=2)
    V_loc = lax.dynamic_slice_in_dim(V2, i * H_PER * Dh, H_PER * Dh, axis=2)

    def body(x_ref, wq_ref, k_ref, v_ref, wo_ref, out_ref,
             comm_ref, recv_ref, send_sems, recv_sems):
        my = lax.axis_index("i")

        barrier_sem = pltpu.get_barrier_semaphore()
        for d in range(1, N_DEV):
            pl.semaphore_signal(
                barrier_sem, inc=1,
                device_id=((my + d) % N_DEV,),
                device_id_type=pl.DeviceIdType.MESH,
            )

        wq = wq_ref[...].astype(jnp.bfloat16)
        wo = wo_ref[...].astype(jnp.bfloat16)
        dn = (((1,), (1,)), ((), ()))

        def compute_unit(b, c):
            rows = slice(c * QB, (c + 1) * QB)
            n_keys = QB if c == 0 else Skv
            xb = x_ref[b, rows, :].astype(jnp.bfloat16)
            q = jnp.dot(xb, wq, preferred_element_type=jnp.float32)
            q = (q * 0.125).astype(jnp.bfloat16)
            kb = k_ref[b, :n_keys, :].astype(jnp.bfloat16)
            vb = v_ref[b, :n_keys, :].astype(jnp.bfloat16)
            ctx_heads = []
            for h in range(H_PER):
                sl = slice(h * Dh, (h + 1) * Dh)
                scores = lax.dot_general(
                    q[:, sl], kb[:, sl], dn,
                    preferred_element_type=jnp.float32,
                )
                w = jnp.exp(scores)
                w = (w / jnp.sum(w, axis=-1, keepdims=True)).astype(jnp.bfloat16)
                ctx_heads.append(
                    jnp.dot(w, vb[:, sl], preferred_element_type=jnp.float32)
                )
            ctx = jnp.concatenate(ctx_heads, axis=1).astype(jnp.bfloat16)
            comm_ref[b, rows, :] = jnp.dot(
                ctx, wo, preferred_element_type=jnp.float32
            ).astype(jnp.bfloat16)

        def make(s, b, c):
            partner = my ^ STEP_MASKS[s]
            rows = pl.ds(c * QB, QB)
            return pltpu.make_async_remote_copy(
                src_ref=comm_ref.at[b, rows, :],
                dst_ref=recv_ref.at[s, b, rows, :],
                send_sem=send_sems.at[s, 2 * b + c],
                recv_sem=recv_sems.at[s, 2 * b + c],
                device_id=(partner,),
                device_id_type=pl.DeviceIdType.MESH,
            )

        rdmas = {}
        compute_unit(0, 0)
        pl.semaphore_wait(barrier_sem, N_DEV - 1)
        rdmas[(0, 0, 0)] = make(0, 0, 0)
        rdmas[(0, 0, 0)].start()
        for b, c in CHUNKS[1:]:
            compute_unit(b, c)
            rdmas[(0, b, c)] = make(0, b, c)
            rdmas[(0, b, c)].start()

        for s in range(N_STEPS):
            for b, c in CHUNKS:
                rows = pl.ds(c * QB, QB)
                rdmas[(s, b, c)].wait()
                comm_ref[b, rows, :] = (
                    comm_ref[b, rows, :] + recv_ref[s, b, rows, :]
                )
                if s + 1 < N_STEPS:
                    rdmas[(s + 1, b, c)] = make(s + 1, b, c)
                    rdmas[(s + 1, b, c)].start()
                else:
                    out_ref[b, rows, :] = comm_ref[b, rows, :].astype(
                        jnp.float32
                    )

    return pl.pallas_call(
        body,
        out_shape=jax.ShapeDtypeStruct((B, Sq, D_MODEL), jnp.float32),
        in_specs=[pl.BlockSpec(memory_space=pltpu.VMEM)] * 5,
        out_specs=pl.BlockSpec(memory_space=pltpu.VMEM),
        scratch_shapes=[
            pltpu.VMEM((B, Sq, D_MODEL), jnp.bfloat16),
            pltpu.VMEM((N_STEPS, B, Sq, D_MODEL), jnp.bfloat16),
            pltpu.SemaphoreType.DMA((N_STEPS, 2 * B)),
            pltpu.SemaphoreType.DMA((N_STEPS, 2 * B)),
        ],
        compiler_params=pltpu.CompilerParams(collective_id=0),
    )(x, Wq, K_loc, V_loc, Wo)


# device time: 24934 ns/iter; 1.0964x vs baseline; 1.0964x over previous
import jax
import jax.numpy as jnp
from jax import lax
from jax.experimental import pallas as pl
from jax.experimental.pallas import tpu as pltpu

N_DEV = 16
RS_MASK = 3
BF_MASKS = [1, 4, 8]
N_STAGES = 1 + len(BF_MASKS)
B, Sq, Skv = 2, 128, 128
H_PER = 4
Dh = 64
D_MODEL = 512
QB = 64


def kernel(x, Wq, K_ext, V_ext, Wo):
    i = lax.axis_index("i")
    K2 = K_ext.reshape(B, Skv, 64 * Dh)
    V2 = V_ext.reshape(B, Skv, 64 * Dh)
    K_loc = lax.dynamic_slice_in_dim(K2, i * H_PER * Dh, H_PER * Dh, axis=2)
    V_loc = lax.dynamic_slice_in_dim(V2, i * H_PER * Dh, H_PER * Dh, axis=2)

    def body(x_ref, wq_ref, k_ref, v_ref, wo_ref, out_ref,
             comm_ref, recv_ref, send_sems, recv_sems):
        my = lax.axis_index("i")
        own_bit = (my >> 1) & 1

        barrier_sem = pltpu.get_barrier_semaphore()
        for d in range(1, N_DEV):
            pl.semaphore_signal(
                barrier_sem, inc=1,
                device_id=((my + d) % N_DEV,),
                device_id_type=pl.DeviceIdType.MESH,
            )

        qi = lax.broadcasted_iota(jnp.int32, (Sq, Skv), 0) // 64
        kj = lax.broadcasted_iota(jnp.int32, (Sq, Skv), 1) // 64
        mask = (qi == kj) | (kj == 0) | ((qi + kj) % 3 == 0)

        wq = wq_ref[...].astype(jnp.bfloat16)
        wo = wo_ref[...].astype(jnp.bfloat16)
        dn = (((1,), (1,)), ((), ()))

        def compute_partial(b):
            xb = x_ref[b].astype(jnp.bfloat16)
            q = jnp.dot(xb, wq, preferred_element_type=jnp.float32)
            q = (q * 0.125).astype(jnp.bfloat16)
            kb = k_ref[b].astype(jnp.bfloat16)
            vb = v_ref[b].astype(jnp.bfloat16)
            ctx_heads = []
            for h in range(H_PER):
                sl = slice(h * Dh, (h + 1) * Dh)
                scores = lax.dot_general(
                    q[:, sl], kb[:, sl], dn,
                    preferred_element_type=jnp.float32,
                )
                scores = jnp.where(mask, scores, -1e9)
                w = jnp.exp(scores)
                w = (w / jnp.sum(w, axis=-1, keepdims=True)).astype(jnp.bfloat16)
                ctx_heads.append(
                    jnp.dot(w, vb[:, sl], preferred_element_type=jnp.float32)
                )
            ctx = jnp.concatenate(ctx_heads, axis=1).astype(jnp.bfloat16)
            comm_ref[b, :, :] = jnp.dot(
                ctx, wo, preferred_element_type=jnp.float32
            ).astype(jnp.bfloat16)

        def pipeline(o):
            g = 1 - o
            rows = [pl.ds(c * QB, QB) for c in range(2)]

            rs = []
            for c in range(2):
                r = pltpu.make_async_remote_copy(
                    src_ref=comm_ref.at[g, rows[c], :],
                    dst_ref=recv_ref.at[0, c],
                    send_sem=send_sems.at[0, c],
                    recv_sem=recv_sems.at[0, c],
                    device_id=(my ^ RS_MASK,),
                    device_id_type=pl.DeviceIdType.MESH,
                )
                r.start()
                rs.append(r)

            compute_partial(o)

            def make_bf(s, c):
                return pltpu.make_async_remote_copy(
                    src_ref=comm_ref.at[o, rows[c], :],
                    dst_ref=recv_ref.at[1 + s, c],
                    send_sem=send_sems.at[1 + s, c],
                    recv_sem=recv_sems.at[1 + s, c],
                    device_id=(my ^ BF_MASKS[s],),
                    device_id_type=pl.DeviceIdType.MESH,
                )

            bf = {}
            for c in range(2):
                rs[c].wait()
                comm_ref[o, rows[c], :] = (
                    comm_ref[o, rows[c], :] + recv_ref[0, c]
                )
                bf[(0, c)] = make_bf(0, c)
                bf[(0, c)].start()

            ag = []
            for s in range(len(BF_MASKS)):
                for c in range(2):
                    bf[(s, c)].wait()
                    comm_ref[o, rows[c], :] = (
                        comm_ref[o, rows[c], :] + recv_ref[1 + s, c]
                    )
                    if s + 1 < len(BF_MASKS):
                        bf[(s + 1, c)] = make_bf(s + 1, c)
                        bf[(s + 1, c)].start()
                    else:
                        r = pltpu.make_async_remote_copy(
                            src_ref=comm_ref.at[o, rows[c], :],
                            dst_ref=comm_ref.at[o, rows[c], :],
                            send_sem=send_sems.at[N_STAGES, c],
                            recv_sem=recv_sems.at[N_STAGES, c],
                            device_id=(my ^ RS_MASK,),
                            device_id_type=pl.DeviceIdType.MESH,
                        )
                        r.start()
                        ag.append(r)
                        out_ref[o, rows[c], :] = comm_ref[
                            o, rows[c], :
                        ].astype(jnp.float32)
            for c in range(2):
                ag[c].wait()
                out_ref[g, rows[c], :] = comm_ref[g, rows[c], :].astype(
                    jnp.float32
                )

        @pl.when(own_bit == 0)
        def _():
            compute_partial(1)

        @pl.when(own_bit == 1)
        def _():
            compute_partial(0)

        pl.semaphore_wait(barrier_sem, N_DEV - 1)

        @pl.when(own_bit == 0)
        def _():
            pipeline(0)

        @pl.when(own_bit == 1)
        def _():
            pipeline(1)

    return pl.pallas_call(
        body,
        out_shape=jax.ShapeDtypeStruct((B, Sq, D_MODEL), jnp.float32),
        in_specs=[pl.BlockSpec(memory_space=pltpu.VMEM)] * 5,
        out_specs=pl.BlockSpec(memory_space=pltpu.VMEM),
        scratch_shapes=[
            pltpu.VMEM((B, Sq, D_MODEL), jnp.bfloat16),
            pltpu.VMEM((N_STAGES, 2, QB, D_MODEL), jnp.bfloat16),
            pltpu.SemaphoreType.DMA((N_STAGES + 1, 2)),
            pltpu.SemaphoreType.DMA((N_STAGES + 1, 2)),
        ],
        compiler_params=pltpu.CompilerParams(collective_id=0),
    )(x, Wq, K_loc, V_loc, Wo)


# device time: 24857 ns/iter; 1.0998x vs baseline; 1.0031x over previous
import jax
import jax.numpy as jnp
from jax import lax
from jax.experimental import pallas as pl
from jax.experimental.pallas import tpu as pltpu

N_DEV = 16
B, Sq, Skv = 2, 128, 128
H_PER = 4
Dh = 64
D_MODEL = 512
QB = 64

P2, P3, P1C0, P1C1, B1C0, B1C1, B2C0, B2C1, AGC0, AGC1 = range(10)
RA, RB, RC, RB1, RB2 = range(5)


def kernel(x, Wq, K_ext, V_ext, Wo):
    i = lax.axis_index("i")
    K2 = K_ext.reshape(B, Skv, 64 * Dh)
    V2 = V_ext.reshape(B, Skv, 64 * Dh)
    K_loc = lax.dynamic_slice_in_dim(K2, i * H_PER * Dh, H_PER * Dh, axis=2)
    V_loc = lax.dynamic_slice_in_dim(V2, i * H_PER * Dh, H_PER * Dh, axis=2)

    def body(x_ref, wq_ref, k_ref, v_ref, wo_ref, out_ref,
             comm_ref, recv_ref, send_sems, recv_sems):
        my = lax.axis_index("i")
        own_bit = (my >> 1) & 1

        barrier_sem = pltpu.get_barrier_semaphore()
        for d in range(1, N_DEV):
            pl.semaphore_signal(
                barrier_sem, inc=1,
                device_id=((my + d) % N_DEV,),
                device_id_type=pl.DeviceIdType.MESH,
            )

        qi = lax.broadcasted_iota(jnp.int32, (Sq, Skv), 0) // 64
        kj = lax.broadcasted_iota(jnp.int32, (Sq, Skv), 1) // 64
        mask = (qi == kj) | (kj == 0) | ((qi + kj) % 3 == 0)

        wq = wq_ref[...].astype(jnp.bfloat16)
        wo = wo_ref[...].astype(jnp.bfloat16)
        dn = (((1,), (1,)), ((), ()))

        def compute_partial(b):
            xb = x_ref[b].astype(jnp.bfloat16)
            q = jnp.dot(xb, wq, preferred_element_type=jnp.float32)
            q = (q * 0.125).astype(jnp.bfloat16)
            kb = k_ref[b].astype(jnp.bfloat16)
            vb = v_ref[b].astype(jnp.bfloat16)
            ctx_heads = []
            for h in range(H_PER):
                sl = slice(h * Dh, (h + 1) * Dh)
                scores = lax.dot_general(
                    q[:, sl], kb[:, sl], dn,
                    preferred_element_type=jnp.float32,
                )
                scores = jnp.where(mask, scores, -1e9)
                w = jnp.exp(scores)
                w = (w / jnp.sum(w, axis=-1, keepdims=True)).astype(jnp.bfloat16)
                ctx_heads.append(
                    jnp.dot(w, vb[:, sl], preferred_element_type=jnp.float32)
                )
            ctx = jnp.concatenate(ctx_heads, axis=1).astype(jnp.bfloat16)
            comm_ref[b, :, :] = jnp.dot(
                ctx, wo, preferred_element_type=jnp.float32
            ).astype(jnp.bfloat16)

        def copy(src, dst, sem_idx, target):
            return pltpu.make_async_remote_copy(
                src_ref=src, dst_ref=dst,
                send_sem=send_sems.at[sem_idx],
                recv_sem=recv_sems.at[sem_idx],
                device_id=(target,),
                device_id_type=pl.DeviceIdType.MESH,
            )

        def pipeline(o):
            g = 1 - o
            rows = [pl.ds(c * QB, QB) for c in range(2)]

            s_p2 = copy(comm_ref.at[g], recv_ref.at[RA], P2, my ^ 2)
            s_p2.start()
            s_p3 = copy(comm_ref.at[g], recv_ref.at[RB], P3, my ^ 3)
            s_p3.start()

            compute_partial(o)

            s_p1 = []
            for c in range(2):
                r = copy(comm_ref.at[o, rows[c], :],
                         recv_ref.at[RC, rows[c], :], P1C0 + c, my ^ 1)
                r.start()
                s_p1.append(r)
            r_a = copy(recv_ref.at[RA], recv_ref.at[RA], P2, my ^ 2)
            r_b = copy(recv_ref.at[RB], recv_ref.at[RB], P3, my ^ 3)

            for r in s_p1:
                r.wait_send()
            r_a.wait_recv()
            comm_ref[o, :, :] = comm_ref[o] + recv_ref[RA]
            r_b.wait_recv()
            comm_ref[o, :, :] = comm_ref[o] + recv_ref[RB]

            bf = {}
            for c in range(2):
                r_c = copy(recv_ref.at[RC, rows[c], :],
                           recv_ref.at[RC, rows[c], :], P1C0 + c, my ^ 1)
                r_c.wait_recv()
                comm_ref[o, rows[c], :] = (
                    comm_ref[o, rows[c], :] + recv_ref[RC, rows[c], :]
                )
                bf[(0, c)] = copy(comm_ref.at[o, rows[c], :],
                                  recv_ref.at[RB1, rows[c], :],
                                  B1C0 + c, my ^ 4)
                bf[(0, c)].start()

            ag = []
            for s in range(2):
                slot = RB1 + s
                for c in range(2):
                    bf[(s, c)].wait()
                    comm_ref[o, rows[c], :] = (
                        comm_ref[o, rows[c], :]
                        + recv_ref[slot, rows[c], :]
                    )
                    if s == 0:
                        bf[(1, c)] = copy(comm_ref.at[o, rows[c], :],
                                          recv_ref.at[RB2, rows[c], :],
                                          B2C0 + c, my ^ 8)
                        bf[(1, c)].start()
                    else:
                        r = copy(comm_ref.at[o, rows[c], :],
                                 comm_ref.at[o, rows[c], :],
                                 AGC0 + c, my ^ 3)
                        r.start()
                        ag.append(r)
                        out_ref[o, rows[c], :] = comm_ref[
                            o, rows[c], :
                        ].astype(jnp.float32)

            s_p2.wait_send()
            s_p3.wait_send()
            for c in range(2):
                ag[c].wait()
                out_ref[g, rows[c], :] = comm_ref[g, rows[c], :].astype(
                    jnp.float32
                )

        @pl.when(own_bit == 0)
        def _():
            compute_partial(1)

        @pl.when(own_bit == 1)
        def _():
            compute_partial(0)

        pl.semaphore_wait(barrier_sem, N_DEV - 1)

        @pl.when(own_bit == 0)
        def _():
            pipeline(0)

        @pl.when(own_bit == 1)
        def _():
            pipeline(1)

    return pl.pallas_call(
        body,
        out_shape=jax.ShapeDtypeStruct((B, Sq, D_MODEL), jnp.float32),
        in_specs=[pl.BlockSpec(memory_space=pltpu.VMEM)] * 5,
        out_specs=pl.BlockSpec(memory_space=pltpu.VMEM),
        scratch_shapes=[
            pltpu.VMEM((B, Sq, D_MODEL), jnp.bfloat16),
            pltpu.VMEM((5, Sq, D_MODEL), jnp.bfloat16),
            pltpu.SemaphoreType.DMA((10,)),
            pltpu.SemaphoreType.DMA((10,)),
        ],
        compiler_params=pltpu.CompilerParams(collective_id=0),
    )(x, Wq, K_loc, V_loc, Wo)
